# baseline (speedup 1.0000x reference)
import functools

import jax
import jax.numpy as jnp
from jax import lax
from jax.experimental import pallas as pl
from jax.experimental.pallas import tpu as pltpu
from jax.experimental.pallas import tpu_sc as plsc

N_CTX = 8192
D_MODEL = 2048
BATCH = 4
NUM_WORKERS = 32
ROWS_PER_WORKER = N_CTX // NUM_WORKERS
CHUNK = 32
NCHUNK = ROWS_PER_WORKER // CHUNK
NSUB = 16


@functools.partial(
    pl.kernel,
    mesh=plsc.VectorSubcoreMesh(core_axis_name="c", subcore_axis_name="s"),
    out_type=jax.ShapeDtypeStruct((BATCH, N_CTX, D_MODEL), jnp.float32),
    scratch_types=[
        pltpu.VMEM((CHUNK, D_MODEL), jnp.float32),
        pltpu.VMEM_SHARED((NSUB * CHUNK, D_MODEL), jnp.float32),
        pltpu.SemaphoreType.DMA,
        pltpu.SemaphoreType.DMA,
    ],
)
def _pos_broadcast(w_hbm, out_hbm, tbuf, spmem, rsem, wsem):
    sub = lax.axis_index("s")
    wid = sub * 2 + lax.axis_index("c")
    base = wid * ROWS_PER_WORKER

    def run(stage_ref):
        def body(i, carry):
            r0 = base + i * CHUNK
            pltpu.async_copy(w_hbm.at[pl.ds(r0, CHUNK)], stage_ref, rsem).wait()
            copies = [
                pltpu.async_copy(stage_ref, out_hbm.at[b, pl.ds(r0, CHUNK)], wsem)
                for b in range(BATCH)
            ]
            for cp in copies:
                cp.wait()
            return carry
        lax.fori_loop(0, NCHUNK, body, 0)

    @pl.when(sub % 2 == 0)
    def _even():
        run(tbuf)

    @pl.when(sub % 2 == 1)
    def _odd():
        run(spmem.at[pl.ds(sub * CHUNK, CHUNK)])


def kernel(tokens, W_pos):
    del tokens
    return _pos_broadcast(W_pos)


# speedup vs baseline: 1.1583x; 1.1583x over previous
import functools

import jax
import jax.numpy as jnp
from jax import lax
from jax.experimental import pallas as pl
from jax.experimental.pallas import tpu as pltpu
from jax.experimental.pallas import tpu_sc as plsc

N_CTX = 8192
D_MODEL = 2048
BATCH = 4
NUM_WORKERS = 32
ROWS_PER_WORKER = N_CTX // NUM_WORKERS
CHUNK = 32
NCHUNK = ROWS_PER_WORKER // CHUNK


@functools.partial(
    pl.kernel,
    mesh=plsc.VectorSubcoreMesh(core_axis_name="c", subcore_axis_name="s"),
    out_type=jax.ShapeDtypeStruct((BATCH, N_CTX, D_MODEL), jnp.float32),
    scratch_types=[
        pltpu.VMEM((CHUNK, D_MODEL), jnp.float32),
        pltpu.SemaphoreType.DMA,
        pltpu.SemaphoreType.DMA,
    ],
)
def _pos_broadcast(w_hbm, out_hbm, buf, rsem, wsem):
    wid = lax.axis_index("s") * 2 + lax.axis_index("c")
    base = wid * ROWS_PER_WORKER

    def body(i, carry):
        r0 = base + i * CHUNK
        pltpu.async_copy(w_hbm.at[pl.ds(r0, CHUNK)], buf, rsem).wait()
        copies = [
            pltpu.async_copy(buf, out_hbm.at[b, pl.ds(r0, CHUNK)], wsem)
            for b in range(BATCH)
        ]
        for cp in copies:
            cp.wait()
        return carry

    lax.fori_loop(0, NCHUNK, body, 0)


def kernel(tokens, W_pos):
    del tokens
    return _pos_broadcast(W_pos)


# speedup vs baseline: 1.1940x; 1.0308x over previous
import functools

import jax
import jax.numpy as jnp
from jax import lax
from jax.experimental import pallas as pl
from jax.experimental.pallas import tpu as pltpu
from jax.experimental.pallas import tpu_sc as plsc

N_CTX = 8192
D_MODEL = 2048
BATCH = 4
NUM_WORKERS = 32
ROWS_PER_WORKER = N_CTX // NUM_WORKERS
CHUNK = 48
_CHUNKS = [(i * CHUNK, CHUNK) for i in range(5)] + [(5 * CHUNK, 16)]


@functools.partial(
    pl.kernel,
    mesh=plsc.VectorSubcoreMesh(core_axis_name="c", subcore_axis_name="s"),
    out_type=jax.ShapeDtypeStruct((BATCH, N_CTX, D_MODEL), jnp.float32),
    scratch_types=[
        pltpu.VMEM((CHUNK, D_MODEL), jnp.float32),
        pltpu.SemaphoreType.DMA,
        pltpu.SemaphoreType.DMA,
    ],
)
def _pos_broadcast(w_hbm, out_hbm, buf, rsem, wsem):
    wid = lax.axis_index("s") * 2 + lax.axis_index("c")
    base = wid * ROWS_PER_WORKER

    for off, n in _CHUNKS:
        r0 = base + off
        pltpu.async_copy(w_hbm.at[pl.ds(r0, n)], buf.at[pl.ds(0, n)], rsem).wait()
        copies = [
            pltpu.async_copy(
                buf.at[pl.ds(0, n)], out_hbm.at[b, pl.ds(r0, n)], wsem)
            for b in range(BATCH)
        ]
        for cp in copies:
            cp.wait()


def kernel(tokens, W_pos):
    del tokens
    return _pos_broadcast(W_pos)


# speedup vs baseline: 1.2037x; 1.0082x over previous
import functools

import jax
import jax.numpy as jnp
from jax import lax
from jax.experimental import pallas as pl
from jax.experimental.pallas import tpu as pltpu
from jax.experimental.pallas import tpu_sc as plsc

N_CTX = 8192
D_MODEL = 2048
BATCH = 4
NUM_WORKERS = 32
ROWS_PER_WORKER = N_CTX // NUM_WORKERS
CHUNK = 56
_CHUNKS = [(i * CHUNK, CHUNK) for i in range(4)] + [(4 * CHUNK, 32)]


@functools.partial(
    pl.kernel,
    mesh=plsc.VectorSubcoreMesh(core_axis_name="c", subcore_axis_name="s"),
    out_type=jax.ShapeDtypeStruct((BATCH, N_CTX, D_MODEL), jnp.float32),
    scratch_types=[
        pltpu.VMEM((CHUNK, D_MODEL), jnp.float32),
        pltpu.SemaphoreType.DMA,
        pltpu.SemaphoreType.DMA,
    ],
)
def _pos_broadcast(w_hbm, out_hbm, buf, rsem, wsem):
    wid = lax.axis_index("s") * 2 + lax.axis_index("c")
    base = wid * ROWS_PER_WORKER

    for off, n in _CHUNKS:
        r0 = base + off
        pltpu.async_copy(w_hbm.at[pl.ds(r0, n)], buf.at[pl.ds(0, n)], rsem).wait()
        copies = [
            pltpu.async_copy(
                buf.at[pl.ds(0, n)], out_hbm.at[b, pl.ds(r0, n)], wsem)
            for b in range(BATCH)
        ]
        for cp in copies:
            cp.wait()


def kernel(tokens, W_pos):
    del tokens
    return _pos_broadcast(W_pos)
